# initial kernel scaffold (unmeasured)
import jax
import jax.numpy as jnp
from jax import lax
from jax.experimental import pallas as pl
from jax.experimental.pallas import tpu as pltpu

try:
    _ds = jax.devices()
    print(f"[probe] n_devices={len(_ds)}", flush=True)
    for _d in _ds:
        print(
            "[probe]",
            _d.id,
            _d.device_kind,
            getattr(_d, "coords", None),
            getattr(_d, "core_on_chip", None),
            flush=True,
        )
except Exception as _e:
    print("[probe] failed:", _e, flush=True)

N_DEV = 8


def kernel(x, w_mat):
    m, k_shard = x.shape
    _, n = w_mat.shape
    m_blk = m // N_DEV

    def body(x_ref, w_ref, out_ref, send_buf, recv_buf, send_sems, recv_sems):
        d = lax.axis_index("i")
        right = lax.rem(d + 1, N_DEV)
        left = lax.rem(d + N_DEV - 1, N_DEV)

        barrier_sem = pltpu.get_barrier_semaphore()

        def nbr_barrier():
            for nbr in (left, right):
                pl.semaphore_signal(
                    barrier_sem,
                    inc=1,
                    device_id=(nbr,),
                    device_id_type=pl.DeviceIdType.MESH,
                )
            pl.semaphore_wait(barrier_sem, 2)

        nbr_barrier()

        for s in range(N_DEV):
            t = lax.rem(d + (N_DEV - 1 - s), N_DEV)
            contrib = jnp.dot(
                x_ref[pl.ds(t * m_blk, m_blk), :],
                w_ref[...],
                preferred_element_type=jnp.float32,
            )
            if s == 0:
                acc = contrib
            else:
                acc = contrib + recv_buf[(s - 1) % 2, :, :].astype(jnp.float32)
            if s < N_DEV - 1:
                send_buf[s % 2, :, :] = acc.astype(jnp.bfloat16)
                rdma = pltpu.make_async_remote_copy(
                    src_ref=send_buf.at[s % 2],
                    dst_ref=recv_buf.at[s % 2],
                    send_sem=send_sems.at[s % 2],
                    recv_sem=recv_sems.at[s % 2],
                    device_id=(right,),
                    device_id_type=pl.DeviceIdType.MESH,
                )
                rdma.start()
                rdma.wait()
                nbr_barrier()
            else:
                out_ref[...] = acc

    return pl.pallas_call(
        body,
        out_shape=jax.ShapeDtypeStruct((m_blk, n), jnp.float32),
        in_specs=[
            pl.BlockSpec(memory_space=pltpu.VMEM),
            pl.BlockSpec(memory_space=pltpu.VMEM),
        ],
        out_specs=pl.BlockSpec(memory_space=pltpu.VMEM),
        scratch_shapes=[
            pltpu.VMEM((2, m_blk, n), jnp.bfloat16),
            pltpu.VMEM((2, m_blk, n), jnp.bfloat16),
            pltpu.SemaphoreType.DMA((2,)),
            pltpu.SemaphoreType.DMA((2,)),
        ],
        compiler_params=pltpu.CompilerParams(collective_id=0),
    )(x, w_mat)


# baseline (device time: 798099 ns/iter reference)
import jax
import jax.numpy as jnp
from jax import lax
from jax.experimental import pallas as pl
from jax.experimental.pallas import tpu as pltpu

N_DEV = 8
N_SUB = 4


def kernel(x, w_mat):
    m, k_shard = x.shape
    _, n = w_mat.shape
    m_blk = m // N_DEV
    n_sub = n // N_SUB

    x = x.astype(jnp.bfloat16)
    w_mat = w_mat.astype(jnp.bfloat16)

    def body(
        x_hbm,
        w_ref,
        out_ref,
        x_blk,
        send_buf,
        recv_buf,
        x_sem,
        send_sem,
        recv_sem,
    ):
        d = lax.axis_index("i")
        right = lax.rem(d + 1, N_DEV)
        left = lax.rem(d + N_DEV - 1, N_DEV)

        barrier_sem = pltpu.get_barrier_semaphore()

        def nbr_barrier():
            for nbr in (left, right):
                pl.semaphore_signal(
                    barrier_sem,
                    inc=1,
                    device_id=(nbr,),
                    device_id_type=pl.DeviceIdType.MESH,
                )
            pl.semaphore_wait(barrier_sem, 2)

        for s in range(N_DEV):
            t = lax.rem(d + (N_DEV - 1 - s), N_DEV)
            slot = s % 2
            fetch = pltpu.make_async_copy(
                x_hbm.at[pl.ds(t * m_blk, m_blk), :],
                x_blk.at[slot],
                x_sem,
            )
            fetch.start()
            fetch.wait()

            def compute_sub(c, _):
                col = pl.ds(c * n_sub, n_sub)
                contrib = jnp.dot(
                    x_blk[slot],
                    w_ref[:, col],
                    preferred_element_type=jnp.float32,
                )
                if s > 0:
                    contrib = contrib + recv_buf[:, col].astype(jnp.float32)
                if s < N_DEV - 1:
                    send_buf[:, col] = contrib.astype(jnp.bfloat16)
                else:
                    out_ref[:, col] = contrib
                return 0

            lax.fori_loop(0, N_SUB, compute_sub, 0)

            if s < N_DEV - 1:
                nbr_barrier()
                rdma = pltpu.make_async_remote_copy(
                    src_ref=send_buf,
                    dst_ref=recv_buf,
                    send_sem=send_sem,
                    recv_sem=recv_sem,
                    device_id=(right,),
                    device_id_type=pl.DeviceIdType.MESH,
                )
                rdma.start()
                rdma.wait()

    return pl.pallas_call(
        body,
        out_shape=jax.ShapeDtypeStruct((m_blk, n), jnp.float32),
        in_specs=[
            pl.BlockSpec(memory_space=pl.ANY),
            pl.BlockSpec(memory_space=pltpu.VMEM),
        ],
        out_specs=pl.BlockSpec(memory_space=pltpu.VMEM),
        scratch_shapes=[
            pltpu.VMEM((2, m_blk, k_shard), jnp.bfloat16),
            pltpu.VMEM((m_blk, n), jnp.bfloat16),
            pltpu.VMEM((m_blk, n), jnp.bfloat16),
            pltpu.SemaphoreType.DMA,
            pltpu.SemaphoreType.DMA,
            pltpu.SemaphoreType.DMA,
        ],
        compiler_params=pltpu.CompilerParams(
            collective_id=0,
            vmem_limit_bytes=60 * 1024 * 1024,
        ),
    )(x, w_mat)


# device time: 375623 ns/iter; 2.1247x vs baseline; 2.1247x over previous
import jax
import jax.numpy as jnp
from jax import lax
from jax.experimental import pallas as pl
from jax.experimental.pallas import tpu as pltpu

N_DEV = 8
SUB = 4


def kernel(x, w_mat):
    m, k_shard = x.shape
    _, n = w_mat.shape
    m_blk = m // N_DEV
    half = n // 2
    sw = half // SUB

    x = x.astype(jnp.bfloat16)
    w_mat = w_mat.astype(jnp.bfloat16)

    def body(
        x_hbm,
        w_ref,
        out_ref,
        x_blk,
        send_buf,
        recv_buf,
        x_sems,
        send_sems,
        recv_sems,
    ):
        d = lax.axis_index("i")
        right = lax.rem(d + 1, N_DEV)
        left = lax.rem(d + N_DEV - 1, N_DEV)
        dir_dst = (right, left)

        barrier_sem = pltpu.get_barrier_semaphore()

        def nbr_barrier():
            for nbr in (left, right):
                pl.semaphore_signal(
                    barrier_sem,
                    inc=1,
                    device_id=(nbr,),
                    device_id_type=pl.DeviceIdType.MESH,
                )
            pl.semaphore_wait(barrier_sem, 2)

        def rdma(dirn, c, parity):
            col = pl.ds(dirn * half + c * sw, sw)
            return pltpu.make_async_remote_copy(
                src_ref=send_buf.at[:, col],
                dst_ref=recv_buf.at[parity, :, col],
                send_sem=send_sems.at[dirn, c, parity],
                recv_sem=recv_sems.at[dirn, c, parity],
                device_id=(dir_dst[dirn],),
                device_id_type=pl.DeviceIdType.MESH,
            )

        for s in range(N_DEV):
            t_cw = lax.rem(d + (N_DEV - 1 - s), N_DEV)
            t_ccw = lax.rem(d + s + 1, N_DEV)
            for dirn, t in ((0, t_cw), (1, t_ccw)):
                pltpu.make_async_copy(
                    x_hbm.at[pl.ds(t * m_blk, m_blk), :],
                    x_blk.at[dirn],
                    x_sems.at[dirn],
                ).start()

            nbr_barrier()

            for dirn in (0, 1):
                pltpu.make_async_copy(
                    x_hbm.at[pl.ds((t_cw if dirn == 0 else t_ccw) * m_blk, m_blk), :],
                    x_blk.at[dirn],
                    x_sems.at[dirn],
                ).wait()

            def sub_body(c, _, s=s):
                for dirn in (0, 1):
                    col = pl.ds(dirn * half + c * sw, sw)
                    if s >= 1:
                        rdma(dirn, c, (s - 1) % 2).wait_send()
                        rdma(dirn, c, (s - 1) % 2).wait_recv()
                    contrib = jnp.dot(
                        x_blk[dirn],
                        w_ref[:, col],
                        preferred_element_type=jnp.float32,
                    )
                    if s >= 1:
                        contrib = contrib + recv_buf[
                            (s - 1) % 2, :, col
                        ].astype(jnp.float32)
                    if s < N_DEV - 1:
                        send_buf[:, col] = contrib.astype(jnp.bfloat16)
                        rdma(dirn, c, s % 2).start()
                    else:
                        out_ref[:, col] = contrib
                return 0

            lax.fori_loop(0, SUB, sub_body, 0)

    return pl.pallas_call(
        body,
        out_shape=jax.ShapeDtypeStruct((m_blk, n), jnp.float32),
        in_specs=[
            pl.BlockSpec(memory_space=pl.ANY),
            pl.BlockSpec(memory_space=pltpu.VMEM),
        ],
        out_specs=pl.BlockSpec(memory_space=pltpu.VMEM),
        scratch_shapes=[
            pltpu.VMEM((2, m_blk, k_shard), jnp.bfloat16),
            pltpu.VMEM((m_blk, n), jnp.bfloat16),
            pltpu.VMEM((2, m_blk, n), jnp.bfloat16),
            pltpu.SemaphoreType.DMA((2,)),
            pltpu.SemaphoreType.DMA((2, SUB, 2)),
            pltpu.SemaphoreType.DMA((2, SUB, 2)),
        ],
        compiler_params=pltpu.CompilerParams(
            collective_id=0,
            vmem_limit_bytes=60 * 1024 * 1024,
        ),
    )(x, w_mat)
